# baseline (speedup 1.0000x reference)
import functools

import jax
import jax.numpy as jnp
from jax import lax
from jax.experimental import pallas as pl
from jax.experimental.pallas import tpu as pltpu
from jax.experimental.pallas import tpu_sc as plsc

NC = 2
NS = 16
NW = NC * NS
B_IDX = 128


def _make_deg_kernel(n_pad, g_per_tile, rpt):
  @functools.partial(
      pl.kernel,
      out_type=jax.ShapeDtypeStruct((NC, n_pad, 16), jnp.float32),
      mesh=plsc.VectorSubcoreMesh(core_axis_name="c", subcore_axis_name="s"),
      scratch_types=[
          pltpu.VMEM((g_per_tile, B_IDX), jnp.int32),
          pltpu.VMEM((B_IDX, 16), jnp.float32),
          pltpu.VMEM_SHARED((n_pad, 16), jnp.float32),
      ],
      compiler_params=pltpu.CompilerParams(use_tc_tiling_on_sc=False),
  )
  def deg_kernel(dstg_hbm, zeros_hbm, ones_hbm, out_hbm, idx_v, ones_v, acc_sh):
    cid = lax.axis_index("c")
    sid = lax.axis_index("s")
    wid = cid * NS + sid
    pltpu.sync_copy(ones_hbm, ones_v)
    pltpu.sync_copy(dstg_hbm.at[pl.ds(wid * g_per_tile, g_per_tile)], idx_v)
    pltpu.sync_copy(zeros_hbm, acc_sh.at[pl.ds(sid * rpt, rpt)])
    plsc.subcore_barrier()

    def body(g, carry):
      pltpu.sync_copy(ones_v, acc_sh.at[idx_v.at[g]], add=True)
      return carry

    lax.fori_loop(0, g_per_tile, body, 0)
    plsc.subcore_barrier()
    pltpu.sync_copy(
        acc_sh.at[pl.ds(sid * rpt, rpt)],
        out_hbm.at[cid].at[pl.ds(sid * rpt, rpt)],
    )

  return deg_kernel


def _make_agg_kernel(n_pad, d, g0, g1, rpt, b_idx):
  ch = 8
  assert g0 % ch == 0 and g1 % ch == 0

  @functools.partial(
      pl.kernel,
      out_type=jax.ShapeDtypeStruct((NC, n_pad, d), jnp.float32),
      mesh=plsc.VectorSubcoreMesh(core_axis_name="c", subcore_axis_name="s"),
      scratch_types=[
          pltpu.VMEM((2, ch, b_idx), jnp.int32),
          pltpu.VMEM((2, ch, b_idx), jnp.int32),
          pltpu.VMEM((2, b_idx, d), jnp.float32),
          pltpu.VMEM_SHARED((n_pad, d), jnp.float32),
          pltpu.SemaphoreType.DMA,
          pltpu.SemaphoreType.DMA,
          pltpu.SemaphoreType.DMA,
      ],
      compiler_params=pltpu.CompilerParams(use_tc_tiling_on_sc=False),
  )
  def agg_kernel(
      table_hbm, srcg_hbm, dstg_hbm, out_hbm, si_v, di_v, rows_v, acc_sh, sem,
      isem, ssem
  ):
    cid = lax.axis_index("c")
    sid = lax.axis_index("s")
    base = jnp.where(cid == 0, sid * g0, NS * g0 + sid * g1)
    ng = jnp.where(cid == 0, g0, g1)

    def zrow(r, carry):
      for cc in range(d // 16):
        rows_v[0, r, pl.ds(cc * 16, 16)] = jnp.zeros((16,), jnp.float32)
      return carry

    lax.fori_loop(0, b_idx, zrow, 0)
    for t in range(rpt // b_idx):
      pltpu.sync_copy(
          rows_v.at[0], acc_sh.at[pl.ds(sid * rpt + t * b_idx, b_idx)]
      )
    plsc.subcore_barrier()

    pltpu.sync_copy(srcg_hbm.at[pl.ds(base, ch)], si_v.at[0])
    pltpu.sync_copy(dstg_hbm.at[pl.ds(base, ch)], di_v.at[0])
    pltpu.async_copy(table_hbm.at[si_v.at[0].at[0]], rows_v.at[0], sem)

    def body(g, carry):
      b = lax.rem(g, 2)
      j = lax.rem(g, ch)
      slot = lax.rem(lax.div(g, ch), 2)
      nslot = lax.rem(slot + 1, 2)
      nx = base + g + ch

      @pl.when(jnp.logical_and(j == 0, g + ch < ng))
      def _():
        pltpu.async_copy(srcg_hbm.at[pl.ds(nx, ch)], si_v.at[nslot], isem)
        pltpu.async_copy(dstg_hbm.at[pl.ds(nx, ch)], di_v.at[nslot], isem)

      pltpu.make_async_copy(
          table_hbm.at[si_v.at[slot].at[j]], rows_v.at[b], sem
      ).wait()

      @pl.when(jnp.logical_and(j == ch - 1, g + 1 < ng))
      def _():
        nx2 = base + g + 1
        pltpu.make_async_copy(
            srcg_hbm.at[pl.ds(nx2, ch)], si_v.at[nslot], isem
        ).wait()
        pltpu.make_async_copy(
            dstg_hbm.at[pl.ds(nx2, ch)], di_v.at[nslot], isem
        ).wait()

      @pl.when(g >= 1)
      def _():
        jp = lax.rem(g - 1, ch)
        slotp = lax.rem(lax.div(g - 1, ch), 2)
        pltpu.make_async_copy(
            rows_v.at[lax.rem(g + 1, 2)],
            acc_sh.at[di_v.at[slotp].at[jp]], ssem
        ).wait()

      @pl.when(g + 1 < ng)
      def _():
        j1 = lax.rem(g + 1, ch)
        slot1 = lax.rem(lax.div(g + 1, ch), 2)
        pltpu.async_copy(
            table_hbm.at[si_v.at[slot1].at[j1]], rows_v.at[lax.rem(g + 1, 2)],
            sem
        )

      pltpu.async_copy(
          rows_v.at[b], acc_sh.at[di_v.at[slot].at[j]], ssem, add=True
      )
      return carry

    lax.fori_loop(0, ng, body, 0)
    jl = lax.rem(ng - 1, ch)
    slotl = lax.rem(lax.div(ng - 1, ch), 2)
    pltpu.make_async_copy(
        rows_v.at[lax.rem(ng - 1, 2)], acc_sh.at[di_v.at[slotl].at[jl]], ssem
    ).wait()
    plsc.subcore_barrier()
    pltpu.sync_copy(
        acc_sh.at[pl.ds(sid * rpt, rpt)],
        out_hbm.at[cid].at[pl.ds(sid * rpt, rpt)],
    )

  return agg_kernel


def _dinv(d0, d1):
  deg = d0[:, 0] + d1[:, 0] + 1.0
  return lax.rsqrt(deg)


def _tc1_body(x_ref, w1_ref, d0_ref, d1_ref, h_ref, hp_ref):
  dinv = _dinv(d0_ref[...], d1_ref[...])
  h = jnp.dot(x_ref[...], w1_ref[...], preferred_element_type=jnp.float32)
  h_ref[...] = h
  hp_ref[...] = h * dinv[:, None]


def _tc2_body(d0_ref, d1_ref, a0_ref, a1_ref, h_ref, b1_ref, w2_ref,
              g_ref, gp_ref):
  dinv = _dinv(d0_ref[...], d1_ref[...])
  out1 = (
      dinv[:, None] * (a0_ref[...] + a1_ref[...])
      + (dinv * dinv)[:, None] * h_ref[...]
      + b1_ref[...]
  )
  g = jnp.dot(out1, w2_ref[...], preferred_element_type=jnp.float32)
  g_ref[...] = g
  gp_ref[...] = g * dinv[:, None]


def _tc3_body(d0_ref, d1_ref, q0_ref, q1_ref, g_ref, b2_ref, out_ref):
  dinv = _dinv(d0_ref[...], d1_ref[...])
  out_ref[...] = (
      dinv[:, None] * (q0_ref[...] + q1_ref[...])
      + (dinv * dinv)[:, None] * g_ref[...]
      + b2_ref[...]
  )


def kernel(x, edge_index, W1, b1, W2, b2):
  n, d_in = x.shape
  d_hid = W1.shape[1]
  n_cls = W2.shape[1]
  e = edge_index.shape[1]

  n_pad = ((n + 1 + 1023) // 1024) * 1024
  rpt = n_pad // NS
  e_grp = NW * B_IDX * 8
  e_pad = ((e + e_grp - 1) // e_grp) * e_grp
  g_per_tile = e_pad // (NW * B_IDX)
  d2 = ((n_cls + 15) // 16) * 16

  src = edge_index[0].astype(jnp.int32)
  dst = edge_index[1].astype(jnp.int32)
  pad = jnp.full((e_pad - e,), n, dtype=jnp.int32)
  src_flat = jnp.concatenate([src, pad])
  dst_flat = jnp.concatenate([dst, pad])
  srcg = src_flat.reshape(e_pad // B_IDX, B_IDX)
  dstg = dst_flat.reshape(e_pad // B_IDX, B_IDX)
  b1x = 64
  srcg1 = src_flat.reshape(e_pad // b1x, b1x)
  dstg1 = dst_flat.reshape(e_pad // b1x, b1x)

  x_pad = jnp.zeros((n_pad, d_in), x.dtype).at[:n].set(x)
  w2_pad = jnp.zeros((d_hid, d2), W2.dtype).at[:, :n_cls].set(W2)
  b1_2d = b1.reshape(1, d_hid)
  b2_2d = jnp.zeros((1, d2), b2.dtype).at[0, :n_cls].set(b2)

  zeros16 = jnp.zeros((rpt, 16), jnp.float32)
  ones16 = jnp.ones((B_IDX, 16), jnp.float32)

  degp = _make_deg_kernel(n_pad, g_per_tile, rpt)(dstg, zeros16, ones16)

  blk = 256
  grid = (n_pad // blk,)
  row_spec = lambda w: pl.BlockSpec((blk, w), lambda i: (i, 0))
  full_spec = lambda a, b: pl.BlockSpec((a, b), lambda i: (0, 0))
  h, hp = pl.pallas_call(
      _tc1_body,
      grid=grid,
      in_specs=[
          row_spec(d_in),
          full_spec(d_in, d_hid),
          row_spec(16),
          row_spec(16),
      ],
      out_specs=[row_spec(d_hid), row_spec(d_hid)],
      out_shape=[
          jax.ShapeDtypeStruct((n_pad, d_hid), jnp.float32),
          jax.ShapeDtypeStruct((n_pad, d_hid), jnp.float32),
      ],
  )(x_pad, W1, degp[0], degp[1])

  tg1 = e_pad // (NS * b1x)
  g1a = 288
  aggp = _make_agg_kernel(n_pad, d_hid, g1a, tg1 - g1a, rpt, b1x)(
      hp, srcg1, dstg1)

  g, gp = pl.pallas_call(
      _tc2_body,
      grid=grid,
      in_specs=[
          row_spec(16),
          row_spec(16),
          row_spec(d_hid),
          row_spec(d_hid),
          row_spec(d_hid),
          full_spec(1, d_hid),
          full_spec(d_hid, d2),
      ],
      out_specs=[row_spec(d2), row_spec(d2)],
      out_shape=[
          jax.ShapeDtypeStruct((n_pad, d2), jnp.float32),
          jax.ShapeDtypeStruct((n_pad, d2), jnp.float32),
      ],
  )(degp[0], degp[1], aggp[0], aggp[1], h, b1_2d, w2_pad)

  tg2 = e_pad // (NS * B_IDX)
  g2a = 136
  qp = _make_agg_kernel(n_pad, d2, g2a, tg2 - g2a, rpt, B_IDX)(gp, srcg, dstg)

  out = pl.pallas_call(
      _tc3_body,
      grid=grid,
      in_specs=[
          row_spec(16),
          row_spec(16),
          row_spec(d2),
          row_spec(d2),
          row_spec(d2),
          full_spec(1, d2),
      ],
      out_specs=row_spec(d2),
      out_shape=jax.ShapeDtypeStruct((n_pad, d2), jnp.float32),
  )(degp[0], degp[1], qp[0], qp[1], g, b2_2d)

  return out[:n, :n_cls]


# speedup vs baseline: 1.1145x; 1.1145x over previous
import functools

import jax
import jax.numpy as jnp
from jax import lax
from jax.experimental import pallas as pl
from jax.experimental.pallas import tpu as pltpu
from jax.experimental.pallas import tpu_sc as plsc

NC = 2
NS = 16
NW = NC * NS
B_IDX = 128


def _make_deg_kernel(n_pad, g_per_tile, rpt):
  @functools.partial(
      pl.kernel,
      out_type=jax.ShapeDtypeStruct((NC, n_pad, 16), jnp.float32),
      mesh=plsc.VectorSubcoreMesh(core_axis_name="c", subcore_axis_name="s"),
      scratch_types=[
          pltpu.VMEM((g_per_tile, B_IDX), jnp.int32),
          pltpu.VMEM((B_IDX, 16), jnp.float32),
          pltpu.VMEM_SHARED((n_pad, 16), jnp.float32),
      ],
      compiler_params=pltpu.CompilerParams(use_tc_tiling_on_sc=False),
  )
  def deg_kernel(dstg_hbm, zeros_hbm, ones_hbm, out_hbm, idx_v, ones_v, acc_sh):
    cid = lax.axis_index("c")
    sid = lax.axis_index("s")
    wid = cid * NS + sid
    pltpu.sync_copy(ones_hbm, ones_v)
    pltpu.sync_copy(dstg_hbm.at[pl.ds(wid * g_per_tile, g_per_tile)], idx_v)
    pltpu.sync_copy(zeros_hbm, acc_sh.at[pl.ds(sid * rpt, rpt)])
    plsc.subcore_barrier()

    def body(g, carry):
      pltpu.sync_copy(ones_v, acc_sh.at[idx_v.at[g]], add=True)
      return carry

    lax.fori_loop(0, g_per_tile, body, 0)
    plsc.subcore_barrier()
    pltpu.sync_copy(
        acc_sh.at[pl.ds(sid * rpt, rpt)],
        out_hbm.at[cid].at[pl.ds(sid * rpt, rpt)],
    )

  return deg_kernel


def _make_agg_kernel(n_pad, d, g0, g1, rpt, b_idx):
  ch = 8
  assert g0 % ch == 0 and g1 % ch == 0

  @functools.partial(
      pl.kernel,
      out_type=jax.ShapeDtypeStruct((NC, n_pad, d), jnp.float32),
      mesh=plsc.VectorSubcoreMesh(core_axis_name="c", subcore_axis_name="s"),
      scratch_types=[
          pltpu.VMEM((2, ch, b_idx), jnp.int32),
          pltpu.VMEM((2, ch, b_idx), jnp.int32),
          pltpu.VMEM((2, b_idx, d), jnp.float32),
          pltpu.VMEM_SHARED((n_pad, d), jnp.float32),
          pltpu.SemaphoreType.DMA,
          pltpu.SemaphoreType.DMA,
          pltpu.SemaphoreType.DMA,
      ],
      compiler_params=pltpu.CompilerParams(use_tc_tiling_on_sc=False),
  )
  def agg_kernel(
      table_hbm, srcg_hbm, dstg_hbm, out_hbm, si_v, di_v, rows_v, acc_sh, sem,
      isem, ssem
  ):
    cid = lax.axis_index("c")
    sid = lax.axis_index("s")
    base = jnp.where(cid == 0, sid * g0, NS * g0 + sid * g1)
    ng = jnp.where(cid == 0, g0, g1)

    def zrow(r, carry):
      for cc in range(d // 16):
        rows_v[0, r, pl.ds(cc * 16, 16)] = jnp.zeros((16,), jnp.float32)
      return carry

    lax.fori_loop(0, b_idx, zrow, 0)
    for t in range(rpt // b_idx):
      pltpu.sync_copy(
          rows_v.at[0], acc_sh.at[pl.ds(sid * rpt + t * b_idx, b_idx)]
      )
    plsc.subcore_barrier()

    pltpu.sync_copy(srcg_hbm.at[pl.ds(base, ch)], si_v.at[0])
    pltpu.sync_copy(dstg_hbm.at[pl.ds(base, ch)], di_v.at[0])
    pltpu.async_copy(table_hbm.at[si_v.at[0].at[0]], rows_v.at[0], sem)

    def body(g, carry):
      b = lax.rem(g, 2)
      j = lax.rem(g, ch)
      slot = lax.rem(lax.div(g, ch), 2)
      nslot = lax.rem(slot + 1, 2)
      nx = base + g + ch

      @pl.when(jnp.logical_and(j == 0, g + ch < ng))
      def _():
        pltpu.async_copy(srcg_hbm.at[pl.ds(nx, ch)], si_v.at[nslot], isem)
        pltpu.async_copy(dstg_hbm.at[pl.ds(nx, ch)], di_v.at[nslot], isem)

      pltpu.make_async_copy(
          table_hbm.at[si_v.at[slot].at[j]], rows_v.at[b], sem
      ).wait()

      @pl.when(jnp.logical_and(j == ch - 1, g + 1 < ng))
      def _():
        nx2 = base + g + 1
        pltpu.make_async_copy(
            srcg_hbm.at[pl.ds(nx2, ch)], si_v.at[nslot], isem
        ).wait()
        pltpu.make_async_copy(
            dstg_hbm.at[pl.ds(nx2, ch)], di_v.at[nslot], isem
        ).wait()

      @pl.when(g >= 1)
      def _():
        jp = lax.rem(g - 1, ch)
        slotp = lax.rem(lax.div(g - 1, ch), 2)
        pltpu.make_async_copy(
            rows_v.at[lax.rem(g + 1, 2)],
            acc_sh.at[di_v.at[slotp].at[jp]], ssem
        ).wait()

      @pl.when(g + 1 < ng)
      def _():
        j1 = lax.rem(g + 1, ch)
        slot1 = lax.rem(lax.div(g + 1, ch), 2)
        pltpu.async_copy(
            table_hbm.at[si_v.at[slot1].at[j1]], rows_v.at[lax.rem(g + 1, 2)],
            sem
        )

      pltpu.async_copy(
          rows_v.at[b], acc_sh.at[di_v.at[slot].at[j]], ssem, add=True
      )
      return carry

    lax.fori_loop(0, ng, body, 0)
    jl = lax.rem(ng - 1, ch)
    slotl = lax.rem(lax.div(ng - 1, ch), 2)
    pltpu.make_async_copy(
        rows_v.at[lax.rem(ng - 1, 2)], acc_sh.at[di_v.at[slotl].at[jl]], ssem
    ).wait()
    plsc.subcore_barrier()
    pltpu.sync_copy(
        acc_sh.at[pl.ds(sid * rpt, rpt)],
        out_hbm.at[cid].at[pl.ds(sid * rpt, rpt)],
    )

  return agg_kernel


def _dinv(d0, d1):
  deg = d0[:, 0] + d1[:, 0] + 1.0
  return lax.rsqrt(deg)


def _tc1_body(x_ref, w1_ref, d0_ref, d1_ref, h_ref, hp_ref):
  dinv = _dinv(d0_ref[...], d1_ref[...])
  h = jnp.dot(x_ref[...], w1_ref[...], preferred_element_type=jnp.float32)
  h_ref[...] = h
  hp_ref[...] = h * dinv[:, None]


def _tc2_body(d0_ref, d1_ref, a0_ref, a1_ref, h_ref, b1_ref, w2_ref,
              g_ref, gp_ref):
  dinv = _dinv(d0_ref[...], d1_ref[...])
  out1 = (
      dinv[:, None] * (a0_ref[...] + a1_ref[...])
      + (dinv * dinv)[:, None] * h_ref[...]
      + b1_ref[...]
  )
  g = jnp.dot(out1, w2_ref[...], preferred_element_type=jnp.float32)
  g_ref[...] = g
  gp_ref[...] = g * dinv[:, None]


def _tc3_body(d0_ref, d1_ref, q0_ref, q1_ref, g_ref, b2_ref, out_ref):
  dinv = _dinv(d0_ref[...], d1_ref[...])
  out_ref[...] = (
      dinv[:, None] * (q0_ref[...] + q1_ref[...])
      + (dinv * dinv)[:, None] * g_ref[...]
      + b2_ref[...]
  )


def kernel(x, edge_index, W1, b1, W2, b2):
  n, d_in = x.shape
  d_hid = W1.shape[1]
  n_cls = W2.shape[1]
  e = edge_index.shape[1]

  n_pad = ((n + 1 + 1023) // 1024) * 1024
  rpt = n_pad // NS
  e_grp = NW * B_IDX * 8
  e_pad = ((e + e_grp - 1) // e_grp) * e_grp
  g_per_tile = e_pad // (NW * B_IDX)
  d2 = ((n_cls + 15) // 16) * 16

  src = edge_index[0].astype(jnp.int32)
  dst = edge_index[1].astype(jnp.int32)
  pad = jnp.full((e_pad - e,), n, dtype=jnp.int32)
  src_flat = jnp.concatenate([src, pad])
  dst_flat = jnp.concatenate([dst, pad])
  srcg = src_flat.reshape(e_pad // B_IDX, B_IDX)
  dstg = dst_flat.reshape(e_pad // B_IDX, B_IDX)
  b1x = 64
  srcg1 = src_flat.reshape(e_pad // b1x, b1x)
  dstg1 = dst_flat.reshape(e_pad // b1x, b1x)

  x_pad = jnp.zeros((n_pad, d_in), x.dtype).at[:n].set(x)
  w2_pad = jnp.zeros((d_hid, d2), W2.dtype).at[:, :n_cls].set(W2)
  b1_2d = b1.reshape(1, d_hid)
  b2_2d = jnp.zeros((1, d2), b2.dtype).at[0, :n_cls].set(b2)

  zeros16 = jnp.zeros((rpt, 16), jnp.float32)
  ones16 = jnp.ones((B_IDX, 16), jnp.float32)

  degp = _make_deg_kernel(n_pad, g_per_tile, rpt)(dstg, zeros16, ones16)

  blk = 512
  grid = (n_pad // blk,)
  row_spec = lambda w: pl.BlockSpec((blk, w), lambda i: (i, 0))
  full_spec = lambda a, b: pl.BlockSpec((a, b), lambda i: (0, 0))
  h, hp = pl.pallas_call(
      _tc1_body,
      grid=grid,
      in_specs=[
          row_spec(d_in),
          full_spec(d_in, d_hid),
          row_spec(16),
          row_spec(16),
      ],
      out_specs=[row_spec(d_hid), row_spec(d_hid)],
      out_shape=[
          jax.ShapeDtypeStruct((n_pad, d_hid), jnp.float32),
          jax.ShapeDtypeStruct((n_pad, d_hid), jnp.float32),
      ],
  )(x_pad, W1, degp[0], degp[1])

  tg1 = e_pad // (NS * b1x)
  g1a = 272
  aggp = _make_agg_kernel(n_pad, d_hid, g1a, tg1 - g1a, rpt, b1x)(
      hp, srcg1, dstg1)

  g, gp = pl.pallas_call(
      _tc2_body,
      grid=grid,
      in_specs=[
          row_spec(16),
          row_spec(16),
          row_spec(d_hid),
          row_spec(d_hid),
          row_spec(d_hid),
          full_spec(1, d_hid),
          full_spec(d_hid, d2),
      ],
      out_specs=[row_spec(d2), row_spec(d2)],
      out_shape=[
          jax.ShapeDtypeStruct((n_pad, d2), jnp.float32),
          jax.ShapeDtypeStruct((n_pad, d2), jnp.float32),
      ],
  )(degp[0], degp[1], aggp[0], aggp[1], h, b1_2d, w2_pad)

  tg2 = e_pad // (NS * B_IDX)
  g2a = 128
  qp = _make_agg_kernel(n_pad, d2, g2a, tg2 - g2a, rpt, B_IDX)(gp, srcg, dstg)

  out = pl.pallas_call(
      _tc3_body,
      grid=grid,
      in_specs=[
          row_spec(16),
          row_spec(16),
          row_spec(d2),
          row_spec(d2),
          row_spec(d2),
          full_spec(1, d2),
      ],
      out_specs=row_spec(d2),
      out_shape=jax.ShapeDtypeStruct((n_pad, d2), jnp.float32),
  )(degp[0], degp[1], qp[0], qp[1], g, b2_2d)

  return out[:n, :n_cls]


# speedup vs baseline: 1.1417x; 1.0244x over previous
import functools

import jax
import jax.numpy as jnp
from jax import lax
from jax.experimental import pallas as pl
from jax.experimental.pallas import tpu as pltpu
from jax.experimental.pallas import tpu_sc as plsc

NC = 2
NS = 16
NW = NC * NS
B_IDX = 128


def _make_deg_kernel(n_pad, g_per_tile, rpt):
  @functools.partial(
      pl.kernel,
      out_type=jax.ShapeDtypeStruct((NC, n_pad, 16), jnp.float32),
      mesh=plsc.VectorSubcoreMesh(core_axis_name="c", subcore_axis_name="s"),
      scratch_types=[
          pltpu.VMEM((g_per_tile, B_IDX), jnp.int32),
          pltpu.VMEM((B_IDX, 16), jnp.float32),
          pltpu.VMEM_SHARED((n_pad, 16), jnp.float32),
      ],
      compiler_params=pltpu.CompilerParams(use_tc_tiling_on_sc=False),
  )
  def deg_kernel(dstg_hbm, zeros_hbm, ones_hbm, out_hbm, idx_v, ones_v, acc_sh):
    cid = lax.axis_index("c")
    sid = lax.axis_index("s")
    wid = cid * NS + sid
    pltpu.sync_copy(ones_hbm, ones_v)
    pltpu.sync_copy(dstg_hbm.at[pl.ds(wid * g_per_tile, g_per_tile)], idx_v)
    pltpu.sync_copy(zeros_hbm, acc_sh.at[pl.ds(sid * rpt, rpt)])
    plsc.subcore_barrier()

    def body(g, carry):
      pltpu.sync_copy(ones_v, acc_sh.at[idx_v.at[g]], add=True)
      return carry

    lax.fori_loop(0, g_per_tile, body, 0)
    plsc.subcore_barrier()
    pltpu.sync_copy(
        acc_sh.at[pl.ds(sid * rpt, rpt)],
        out_hbm.at[cid].at[pl.ds(sid * rpt, rpt)],
    )

  return deg_kernel


def _make_agg_kernel(n_pad, d, g0, g1, rpt, b_idx):
  ch = 8
  assert g0 % ch == 0 and g1 % ch == 0

  @functools.partial(
      pl.kernel,
      out_type=jax.ShapeDtypeStruct((NC, n_pad, d), jnp.float32),
      mesh=plsc.VectorSubcoreMesh(core_axis_name="c", subcore_axis_name="s"),
      scratch_types=[
          pltpu.VMEM((2, ch, b_idx), jnp.int32),
          pltpu.VMEM((2, ch, b_idx), jnp.int32),
          pltpu.VMEM((2, b_idx, d), jnp.float32),
          pltpu.VMEM_SHARED((n_pad, d), jnp.float32),
          pltpu.SemaphoreType.DMA,
          pltpu.SemaphoreType.DMA,
          pltpu.SemaphoreType.DMA,
      ],
      compiler_params=pltpu.CompilerParams(use_tc_tiling_on_sc=False),
  )
  def agg_kernel(
      table_hbm, srcg_hbm, dstg_hbm, out_hbm, si_v, di_v, rows_v, acc_sh, sem,
      isem, ssem
  ):
    cid = lax.axis_index("c")
    sid = lax.axis_index("s")
    base = jnp.where(cid == 0, sid * g0, NS * g0 + sid * g1)
    ng = jnp.where(cid == 0, g0, g1)

    def zrow(r, carry):
      for cc in range(d // 16):
        rows_v[0, r, pl.ds(cc * 16, 16)] = jnp.zeros((16,), jnp.float32)
      return carry

    lax.fori_loop(0, b_idx, zrow, 0)
    for t in range(rpt // b_idx):
      pltpu.sync_copy(
          rows_v.at[0], acc_sh.at[pl.ds(sid * rpt + t * b_idx, b_idx)]
      )
    plsc.subcore_barrier()

    pltpu.sync_copy(srcg_hbm.at[pl.ds(base, ch)], si_v.at[0])
    pltpu.sync_copy(dstg_hbm.at[pl.ds(base, ch)], di_v.at[0])
    pltpu.async_copy(table_hbm.at[si_v.at[0].at[0]], rows_v.at[0], sem)

    def body(g, carry):
      b = lax.rem(g, 2)
      j = lax.rem(g, ch)
      slot = lax.rem(lax.div(g, ch), 2)
      nslot = lax.rem(slot + 1, 2)
      nx = base + g + ch

      @pl.when(jnp.logical_and(j == 0, g + ch < ng))
      def _():
        pltpu.async_copy(srcg_hbm.at[pl.ds(nx, ch)], si_v.at[nslot], isem)
        pltpu.async_copy(dstg_hbm.at[pl.ds(nx, ch)], di_v.at[nslot], isem)

      pltpu.make_async_copy(
          table_hbm.at[si_v.at[slot].at[j]], rows_v.at[b], sem
      ).wait()

      @pl.when(jnp.logical_and(j == ch - 1, g + 1 < ng))
      def _():
        nx2 = base + g + 1
        pltpu.make_async_copy(
            srcg_hbm.at[pl.ds(nx2, ch)], si_v.at[nslot], isem
        ).wait()
        pltpu.make_async_copy(
            dstg_hbm.at[pl.ds(nx2, ch)], di_v.at[nslot], isem
        ).wait()

      @pl.when(g >= 1)
      def _():
        jp = lax.rem(g - 1, ch)
        slotp = lax.rem(lax.div(g - 1, ch), 2)
        pltpu.make_async_copy(
            rows_v.at[lax.rem(g + 1, 2)],
            acc_sh.at[di_v.at[slotp].at[jp]], ssem
        ).wait()

      @pl.when(g + 1 < ng)
      def _():
        j1 = lax.rem(g + 1, ch)
        slot1 = lax.rem(lax.div(g + 1, ch), 2)
        pltpu.async_copy(
            table_hbm.at[si_v.at[slot1].at[j1]], rows_v.at[lax.rem(g + 1, 2)],
            sem
        )

      pltpu.async_copy(
          rows_v.at[b], acc_sh.at[di_v.at[slot].at[j]], ssem, add=True
      )
      return carry

    lax.fori_loop(0, ng, body, 0)
    jl = lax.rem(ng - 1, ch)
    slotl = lax.rem(lax.div(ng - 1, ch), 2)
    pltpu.make_async_copy(
        rows_v.at[lax.rem(ng - 1, 2)], acc_sh.at[di_v.at[slotl].at[jl]], ssem
    ).wait()
    plsc.subcore_barrier()
    pltpu.sync_copy(
        acc_sh.at[pl.ds(sid * rpt, rpt)],
        out_hbm.at[cid].at[pl.ds(sid * rpt, rpt)],
    )

  return agg_kernel


def _dinv(d0, d1):
  deg = d0[:, 0] + d1[:, 0] + 1.0
  return lax.rsqrt(deg)


def _tc1_body(x_ref, w1_ref, d0_ref, d1_ref, h_ref, hp_ref):
  dinv = _dinv(d0_ref[...], d1_ref[...])
  h = jnp.dot(x_ref[...], w1_ref[...], preferred_element_type=jnp.float32)
  h_ref[...] = h
  hp_ref[...] = h * dinv[:, None]


def _tc2_body(d0_ref, d1_ref, a0_ref, a1_ref, h_ref, b1_ref, w2_ref,
              g_ref, gp_ref):
  dinv = _dinv(d0_ref[...], d1_ref[...])
  out1 = (
      dinv[:, None] * (a0_ref[...] + a1_ref[...])
      + (dinv * dinv)[:, None] * h_ref[...]
      + b1_ref[...]
  )
  g = jnp.dot(out1, w2_ref[...], preferred_element_type=jnp.float32)
  g_ref[...] = g
  gp_ref[...] = g * dinv[:, None]


def _tc3_body(d0_ref, d1_ref, q0_ref, q1_ref, g_ref, b2_ref, out_ref):
  dinv = _dinv(d0_ref[...], d1_ref[...])
  out_ref[...] = (
      dinv[:, None] * (q0_ref[...] + q1_ref[...])
      + (dinv * dinv)[:, None] * g_ref[...]
      + b2_ref[...]
  )


def kernel(x, edge_index, W1, b1, W2, b2):
  n, d_in = x.shape
  d_hid = W1.shape[1]
  n_cls = W2.shape[1]
  e = edge_index.shape[1]

  n_pad = ((n + 1 + 1023) // 1024) * 1024
  rpt = n_pad // NS
  e_grp = NW * B_IDX * 8
  e_pad = ((e + e_grp - 1) // e_grp) * e_grp
  g_per_tile = e_pad // (NW * B_IDX)
  d2 = ((n_cls + 15) // 16) * 16

  src = edge_index[0].astype(jnp.int32)
  dst = edge_index[1].astype(jnp.int32)
  pad = jnp.full((e_pad - e,), n, dtype=jnp.int32)
  src_flat = jnp.concatenate([src, pad])
  dst_flat = jnp.concatenate([dst, pad])
  srcg = src_flat.reshape(e_pad // B_IDX, B_IDX)
  dstg = dst_flat.reshape(e_pad // B_IDX, B_IDX)
  b1x = 64
  srcg1 = src_flat.reshape(e_pad // b1x, b1x)
  dstg1 = dst_flat.reshape(e_pad // b1x, b1x)

  x_pad = jnp.zeros((n_pad, d_in), x.dtype).at[:n].set(x)
  w2_pad = jnp.zeros((d_hid, d2), W2.dtype).at[:, :n_cls].set(W2)
  b1_2d = b1.reshape(1, d_hid)
  b2_2d = jnp.zeros((1, d2), b2.dtype).at[0, :n_cls].set(b2)

  zeros16 = jnp.zeros((rpt, 16), jnp.float32)
  ones16 = jnp.ones((B_IDX, 16), jnp.float32)

  degp = _make_deg_kernel(n_pad, g_per_tile, rpt)(dstg, zeros16, ones16)

  blk = 1024
  grid = (n_pad // blk,)
  row_spec = lambda w: pl.BlockSpec((blk, w), lambda i: (i, 0))
  full_spec = lambda a, b: pl.BlockSpec((a, b), lambda i: (0, 0))
  h, hp = pl.pallas_call(
      _tc1_body,
      grid=grid,
      in_specs=[
          row_spec(d_in),
          full_spec(d_in, d_hid),
          row_spec(16),
          row_spec(16),
      ],
      out_specs=[row_spec(d_hid), row_spec(d_hid)],
      out_shape=[
          jax.ShapeDtypeStruct((n_pad, d_hid), jnp.float32),
          jax.ShapeDtypeStruct((n_pad, d_hid), jnp.float32),
      ],
  )(x_pad, W1, degp[0], degp[1])

  tg1 = e_pad // (NS * b1x)
  g1a = 272
  aggp = _make_agg_kernel(n_pad, d_hid, g1a, tg1 - g1a, rpt, b1x)(
      hp, srcg1, dstg1)

  g, gp = pl.pallas_call(
      _tc2_body,
      grid=grid,
      in_specs=[
          row_spec(16),
          row_spec(16),
          row_spec(d_hid),
          row_spec(d_hid),
          row_spec(d_hid),
          full_spec(1, d_hid),
          full_spec(d_hid, d2),
      ],
      out_specs=[row_spec(d2), row_spec(d2)],
      out_shape=[
          jax.ShapeDtypeStruct((n_pad, d2), jnp.float32),
          jax.ShapeDtypeStruct((n_pad, d2), jnp.float32),
      ],
  )(degp[0], degp[1], aggp[0], aggp[1], h, b1_2d, w2_pad)

  tg2 = e_pad // (NS * B_IDX)
  g2a = 128
  qp = _make_agg_kernel(n_pad, d2, g2a, tg2 - g2a, rpt, B_IDX)(gp, srcg, dstg)

  out = pl.pallas_call(
      _tc3_body,
      grid=grid,
      in_specs=[
          row_spec(16),
          row_spec(16),
          row_spec(d2),
          row_spec(d2),
          row_spec(d2),
          full_spec(1, d2),
      ],
      out_specs=row_spec(d2),
      out_shape=jax.ShapeDtypeStruct((n_pad, d2), jnp.float32),
  )(degp[0], degp[1], qp[0], qp[1], g, b2_2d)

  return out[:n, :n_cls]


# speedup vs baseline: 1.1496x; 1.0069x over previous
import functools

import jax
import jax.numpy as jnp
from jax import lax
from jax.experimental import pallas as pl
from jax.experimental.pallas import tpu as pltpu
from jax.experimental.pallas import tpu_sc as plsc

NC = 2
NS = 16
NW = NC * NS
B_IDX = 128


def _make_deg_kernel(n_pad, g_per_tile, rpt):
  @functools.partial(
      pl.kernel,
      out_type=jax.ShapeDtypeStruct((NC, n_pad, 16), jnp.float32),
      mesh=plsc.VectorSubcoreMesh(core_axis_name="c", subcore_axis_name="s"),
      scratch_types=[
          pltpu.VMEM((g_per_tile, B_IDX), jnp.int32),
          pltpu.VMEM((B_IDX, 16), jnp.float32),
          pltpu.VMEM_SHARED((n_pad, 16), jnp.float32),
      ],
      compiler_params=pltpu.CompilerParams(use_tc_tiling_on_sc=False),
  )
  def deg_kernel(dstg_hbm, zeros_hbm, ones_hbm, out_hbm, idx_v, ones_v, acc_sh):
    cid = lax.axis_index("c")
    sid = lax.axis_index("s")
    wid = cid * NS + sid
    pltpu.sync_copy(ones_hbm, ones_v)
    pltpu.sync_copy(dstg_hbm.at[pl.ds(wid * g_per_tile, g_per_tile)], idx_v)
    pltpu.sync_copy(zeros_hbm, acc_sh.at[pl.ds(sid * rpt, rpt)])
    plsc.subcore_barrier()

    def body(g, carry):
      pltpu.sync_copy(ones_v, acc_sh.at[idx_v.at[g]], add=True)
      return carry

    lax.fori_loop(0, g_per_tile, body, 0)
    plsc.subcore_barrier()
    pltpu.sync_copy(
        acc_sh.at[pl.ds(sid * rpt, rpt)],
        out_hbm.at[cid].at[pl.ds(sid * rpt, rpt)],
    )

  return deg_kernel


def _make_agg_kernel(n_pad, d, g0, g1, rpt, b_idx):
  ch = 8
  assert g0 % ch == 0 and g1 % ch == 0

  @functools.partial(
      pl.kernel,
      out_type=jax.ShapeDtypeStruct((NC, n_pad, d), jnp.float32),
      mesh=plsc.VectorSubcoreMesh(core_axis_name="c", subcore_axis_name="s"),
      scratch_types=[
          pltpu.VMEM((2, ch, b_idx), jnp.int32),
          pltpu.VMEM((2, ch, b_idx), jnp.int32),
          pltpu.VMEM((2, b_idx, d), jnp.float32),
          pltpu.VMEM_SHARED((n_pad, d), jnp.float32),
          pltpu.SemaphoreType.DMA,
          pltpu.SemaphoreType.DMA,
          pltpu.SemaphoreType.DMA,
      ],
      compiler_params=pltpu.CompilerParams(use_tc_tiling_on_sc=False),
  )
  def agg_kernel(
      table_hbm, srcg_hbm, dstg_hbm, out_hbm, si_v, di_v, rows_v, acc_sh, sem,
      isem, ssem
  ):
    cid = lax.axis_index("c")
    sid = lax.axis_index("s")
    base = jnp.where(cid == 0, sid * g0, NS * g0 + sid * g1)
    ng = jnp.where(cid == 0, g0, g1)

    def zrow(r, carry):
      for cc in range(d // 16):
        rows_v[0, r, pl.ds(cc * 16, 16)] = jnp.zeros((16,), jnp.float32)
      return carry

    lax.fori_loop(0, b_idx, zrow, 0)
    for t in range(rpt // b_idx):
      pltpu.sync_copy(
          rows_v.at[0], acc_sh.at[pl.ds(sid * rpt + t * b_idx, b_idx)]
      )
    plsc.subcore_barrier()

    pltpu.sync_copy(srcg_hbm.at[pl.ds(base, ch)], si_v.at[0])
    pltpu.sync_copy(dstg_hbm.at[pl.ds(base, ch)], di_v.at[0])
    pltpu.async_copy(table_hbm.at[si_v.at[0].at[0]], rows_v.at[0], sem)

    def body(g, carry):
      b = lax.rem(g, 2)
      j = lax.rem(g, ch)
      slot = lax.rem(lax.div(g, ch), 2)
      nslot = lax.rem(slot + 1, 2)
      nx = base + g + ch

      @pl.when(jnp.logical_and(j == 0, g + ch < ng))
      def _():
        pltpu.async_copy(srcg_hbm.at[pl.ds(nx, ch)], si_v.at[nslot], isem)
        pltpu.async_copy(dstg_hbm.at[pl.ds(nx, ch)], di_v.at[nslot], isem)

      pltpu.make_async_copy(
          table_hbm.at[si_v.at[slot].at[j]], rows_v.at[b], sem
      ).wait()

      @pl.when(jnp.logical_and(j == ch - 1, g + 1 < ng))
      def _():
        nx2 = base + g + 1
        pltpu.make_async_copy(
            srcg_hbm.at[pl.ds(nx2, ch)], si_v.at[nslot], isem
        ).wait()
        pltpu.make_async_copy(
            dstg_hbm.at[pl.ds(nx2, ch)], di_v.at[nslot], isem
        ).wait()

      @pl.when(g >= 1)
      def _():
        jp = lax.rem(g - 1, ch)
        slotp = lax.rem(lax.div(g - 1, ch), 2)
        pltpu.make_async_copy(
            rows_v.at[lax.rem(g + 1, 2)],
            acc_sh.at[di_v.at[slotp].at[jp]], ssem
        ).wait()

      @pl.when(g + 1 < ng)
      def _():
        j1 = lax.rem(g + 1, ch)
        slot1 = lax.rem(lax.div(g + 1, ch), 2)
        pltpu.async_copy(
            table_hbm.at[si_v.at[slot1].at[j1]], rows_v.at[lax.rem(g + 1, 2)],
            sem
        )

      pltpu.async_copy(
          rows_v.at[b], acc_sh.at[di_v.at[slot].at[j]], ssem, add=True
      )
      return carry

    lax.fori_loop(0, ng, body, 0)
    jl = lax.rem(ng - 1, ch)
    slotl = lax.rem(lax.div(ng - 1, ch), 2)
    pltpu.make_async_copy(
        rows_v.at[lax.rem(ng - 1, 2)], acc_sh.at[di_v.at[slotl].at[jl]], ssem
    ).wait()
    plsc.subcore_barrier()
    pltpu.sync_copy(
        acc_sh.at[pl.ds(sid * rpt, rpt)],
        out_hbm.at[cid].at[pl.ds(sid * rpt, rpt)],
    )

  return agg_kernel


def _dinv(d0, d1):
  deg = d0[:, 0] + d1[:, 0] + 1.0
  return lax.rsqrt(deg)


def _tc1_body(x_ref, w1_ref, d0_ref, d1_ref, h_ref, hp_ref):
  dinv = _dinv(d0_ref[...], d1_ref[...])
  h = jnp.dot(x_ref[...], w1_ref[...], preferred_element_type=jnp.float32)
  h_ref[...] = h
  hp_ref[...] = h * dinv[:, None]


def _tc2_body(d0_ref, d1_ref, a0_ref, a1_ref, h_ref, b1_ref, w2_ref,
              g_ref, gp_ref):
  dinv = _dinv(d0_ref[...], d1_ref[...])
  out1 = (
      dinv[:, None] * (a0_ref[...] + a1_ref[...])
      + (dinv * dinv)[:, None] * h_ref[...]
      + b1_ref[...]
  )
  g = jnp.dot(out1, w2_ref[...], preferred_element_type=jnp.float32)
  g_ref[...] = g
  gp_ref[...] = g * dinv[:, None]


def _tc3_body(d0_ref, d1_ref, q0_ref, q1_ref, g_ref, b2_ref, out_ref):
  dinv = _dinv(d0_ref[...], d1_ref[...])
  out_ref[...] = (
      dinv[:, None] * (q0_ref[...] + q1_ref[...])
      + (dinv * dinv)[:, None] * g_ref[...]
      + b2_ref[...]
  )


def kernel(x, edge_index, W1, b1, W2, b2):
  n, d_in = x.shape
  d_hid = W1.shape[1]
  n_cls = W2.shape[1]
  e = edge_index.shape[1]

  n_pad = ((n + 1 + 1023) // 1024) * 1024
  rpt = n_pad // NS
  e_grp = NW * B_IDX * 8
  e_pad = ((e + e_grp - 1) // e_grp) * e_grp
  g_per_tile = e_pad // (NW * B_IDX)
  d2 = ((n_cls + 15) // 16) * 16

  src = edge_index[0].astype(jnp.int32)
  dst = edge_index[1].astype(jnp.int32)
  pad = jnp.full((e_pad - e,), n, dtype=jnp.int32)
  src_flat = jnp.concatenate([src, pad])
  dst_flat = jnp.concatenate([dst, pad])
  srcg = src_flat.reshape(e_pad // B_IDX, B_IDX)
  dstg = dst_flat.reshape(e_pad // B_IDX, B_IDX)
  b1x = 64
  srcg1 = src_flat.reshape(e_pad // b1x, b1x)
  dstg1 = dst_flat.reshape(e_pad // b1x, b1x)

  x_pad = jnp.zeros((n_pad, d_in), x.dtype).at[:n].set(x)
  w2_pad = jnp.zeros((d_hid, d2), W2.dtype).at[:, :n_cls].set(W2)
  b1_2d = b1.reshape(1, d_hid)
  b2_2d = jnp.zeros((1, d2), b2.dtype).at[0, :n_cls].set(b2)

  zeros16 = jnp.zeros((rpt, 16), jnp.float32)
  ones16 = jnp.ones((B_IDX, 16), jnp.float32)

  degp = _make_deg_kernel(n_pad, g_per_tile, rpt)(dstg, zeros16, ones16)

  blk = 2048
  grid = (n_pad // blk,)
  row_spec = lambda w: pl.BlockSpec((blk, w), lambda i: (i, 0))
  full_spec = lambda a, b: pl.BlockSpec((a, b), lambda i: (0, 0))
  h, hp = pl.pallas_call(
      _tc1_body,
      grid=grid,
      in_specs=[
          row_spec(d_in),
          full_spec(d_in, d_hid),
          row_spec(16),
          row_spec(16),
      ],
      out_specs=[row_spec(d_hid), row_spec(d_hid)],
      out_shape=[
          jax.ShapeDtypeStruct((n_pad, d_hid), jnp.float32),
          jax.ShapeDtypeStruct((n_pad, d_hid), jnp.float32),
      ],
  )(x_pad, W1, degp[0], degp[1])

  tg1 = e_pad // (NS * b1x)
  g1a = 272
  aggp = _make_agg_kernel(n_pad, d_hid, g1a, tg1 - g1a, rpt, b1x)(
      hp, srcg1, dstg1)

  g, gp = pl.pallas_call(
      _tc2_body,
      grid=grid,
      in_specs=[
          row_spec(16),
          row_spec(16),
          row_spec(d_hid),
          row_spec(d_hid),
          row_spec(d_hid),
          full_spec(1, d_hid),
          full_spec(d_hid, d2),
      ],
      out_specs=[row_spec(d2), row_spec(d2)],
      out_shape=[
          jax.ShapeDtypeStruct((n_pad, d2), jnp.float32),
          jax.ShapeDtypeStruct((n_pad, d2), jnp.float32),
      ],
  )(degp[0], degp[1], aggp[0], aggp[1], h, b1_2d, w2_pad)

  tg2 = e_pad // (NS * B_IDX)
  g2a = 128
  qp = _make_agg_kernel(n_pad, d2, g2a, tg2 - g2a, rpt, B_IDX)(gp, srcg, dstg)

  out = pl.pallas_call(
      _tc3_body,
      grid=grid,
      in_specs=[
          row_spec(16),
          row_spec(16),
          row_spec(d2),
          row_spec(d2),
          row_spec(d2),
          full_spec(1, d2),
      ],
      out_specs=row_spec(d2),
      out_shape=jax.ShapeDtypeStruct((n_pad, d2), jnp.float32),
  )(degp[0], degp[1], qp[0], qp[1], g, b2_2d)

  return out[:n, :n_cls]
